# baseline (speedup 1.0000x reference)
import jax
import jax.numpy as jnp
from jax.experimental import pallas as pl
from jax.experimental.pallas import tpu as pltpu

N = 4096
D = 128
TILE_R = 2048
TILE_C = 1024
GRID_R = N // TILE_R
GRID_C = N // TILE_C
NSTEPS = GRID_R * GRID_C


def _masked_block(zi_ref, zj_ref, kr_ref, kc_ref, i):
    gram = jax.lax.dot_general(
        zi_ref[...], zj_ref[...],
        dimension_numbers=(((1,), (1,)), ((), ())),
        preferred_element_type=jnp.float32,
    )
    mask = kr_ref[...] == kc_ref[...]
    return jnp.where(mask, gram, 0.0)


def _fix_diag(acc_ref, i, j):
    off = i * TILE_R - j * TILE_C

    @pl.when((off > -TILE_R) & (off < TILE_C))
    def _():
        eye = (jax.lax.broadcasted_iota(jnp.int32, (128, 128), 0)
               == jax.lax.broadcasted_iota(jnp.int32, (128, 128), 1))
        for k in range(TILE_R // 128):
            c0 = k * 128 + off

            @pl.when((c0 >= 0) & (c0 < TILE_C))
            def _blk():
                rs = pl.ds(k * 128, 128)
                cs = pl.ds(c0, 128)
                acc_ref[rs, cs] = jnp.where(eye, 0.0, acc_ref[rs, cs])


def _seg_kernel(sr_ref, sc_ref, zi_ref, zj_ref, kr_ref, kc_ref, out_ref,
                acc0_ref, acc1_ref, zbuf_ref, sems):
    step = pl.program_id(0)
    i = step // GRID_C
    j = step % GRID_C
    slot = jax.lax.rem(step, 2)

    @pl.when(step == 0)
    def _init_zeros():
        zbuf_ref[...] = jnp.zeros((TILE_R, TILE_C), jnp.float32)

    dst = out_ref.at[pl.ds(i * TILE_R, TILE_R), pl.ds(j * TILE_C, TILE_C)]

    @pl.when(step >= 2)
    def _wait_prev():
        pltpu.make_async_copy(acc0_ref, dst, sems.at[step - 2]).wait()

    r_lo = sr_ref[0, i]
    r_hi = sr_ref[1, i]
    c_lo = sc_ref[0, j]
    c_hi = sc_ref[1, j]
    overlap = (r_hi >= c_lo) & (c_hi >= r_lo)

    @pl.when(overlap)
    def _compute():
        masked = _masked_block(zi_ref, zj_ref, kr_ref, kc_ref, i)

        @pl.when(slot == 0)
        def _s0():
            acc0_ref[...] = masked
            _fix_diag(acc0_ref, i, j)
            pltpu.make_async_copy(acc0_ref, dst, sems.at[step]).start()

        @pl.when(slot == 1)
        def _s1():
            acc1_ref[...] = masked
            _fix_diag(acc1_ref, i, j)
            pltpu.make_async_copy(acc1_ref, dst, sems.at[step]).start()

    @pl.when(~overlap)
    def _zero():
        pltpu.make_async_copy(zbuf_ref, dst, sems.at[step]).start()

    @pl.when(step == NSTEPS - 1)
    def _drain():
        pltpu.make_async_copy(acc0_ref, dst, sems.at[NSTEPS - 2]).wait()
        pltpu.make_async_copy(acc0_ref, dst, sems.at[NSTEPS - 1]).wait()


def kernel(z, cls_label, batch):
    valid = ~((cls_label == 24) | (cls_label == 25) | (cls_label == 26))
    key = batch * 64 + cls_label
    key_row = jnp.where(valid, key, -1).reshape(N, 1)
    key_col = jnp.where(valid, key, -2).reshape(1, N)
    ep_row = jnp.stack([batch[::TILE_R], batch[TILE_R - 1::TILE_R]])
    ep_col = jnp.stack([batch[::TILE_C], batch[TILE_C - 1::TILE_C]])
    grid_spec = pltpu.PrefetchScalarGridSpec(
        num_scalar_prefetch=2,
        grid=(NSTEPS,),
        in_specs=[
            pl.BlockSpec((TILE_R, D), lambda s, sr, sc: (s // GRID_C, 0)),
            pl.BlockSpec((TILE_C, D), lambda s, sr, sc: (s % GRID_C, 0)),
            pl.BlockSpec((TILE_R, 1), lambda s, sr, sc: (s // GRID_C, 0)),
            pl.BlockSpec((1, TILE_C), lambda s, sr, sc: (0, s % GRID_C)),
        ],
        out_specs=pl.BlockSpec(memory_space=pl.ANY),
        scratch_shapes=[
            pltpu.VMEM((TILE_R, TILE_C), jnp.float32),
            pltpu.VMEM((TILE_R, TILE_C), jnp.float32),
            pltpu.VMEM((TILE_R, TILE_C), jnp.float32),
            pltpu.SemaphoreType.DMA((NSTEPS,)),
        ],
    )
    return pl.pallas_call(
        _seg_kernel,
        grid_spec=grid_spec,
        out_shape=jax.ShapeDtypeStruct((N, N), jnp.float32),
    )(ep_row, ep_col, z, z, key_row, key_col)


# speedup vs baseline: 1.0567x; 1.0567x over previous
import jax
import jax.numpy as jnp
from jax.experimental import pallas as pl
from jax.experimental.pallas import tpu as pltpu

N = 4096
D = 128
TILE_R = 2048
TILE_C = 2048
GRID_R = N // TILE_R
GRID_C = N // TILE_C
NSTEPS = GRID_R * GRID_C


def _overlap(sr_ref, sc_ref, step):
    i = step // GRID_C
    j = step % GRID_C
    return (sr_ref[1, i] >= sc_ref[0, j]) & (sc_ref[1, j] >= sr_ref[0, i])


def _fix_diag(acc_ref, i, j):
    off = i * TILE_R - j * TILE_C

    @pl.when((off > -TILE_R) & (off < TILE_C))
    def _():
        eye = (jax.lax.broadcasted_iota(jnp.int32, (128, 128), 0)
               == jax.lax.broadcasted_iota(jnp.int32, (128, 128), 1))
        for k in range(TILE_R // 128):
            c0 = k * 128 + off

            @pl.when((c0 >= 0) & (c0 < TILE_C))
            def _blk():
                rs = pl.ds(k * 128, 128)
                cs = pl.ds(c0, 128)
                acc_ref[rs, cs] = jnp.where(eye, 0.0, acc_ref[rs, cs])


def _seg_kernel(sr_ref, sc_ref, zi_ref, zj_ref, kr_ref, kc_ref, out_ref,
                acc0_ref, acc1_ref, zbuf_ref, sems):
    step = pl.program_id(0)
    i = step // GRID_C
    j = step % GRID_C
    slot = jax.lax.rem(step, 2)

    @pl.when(step == 0)
    def _init_zeros():
        zbuf_ref[...] = jnp.zeros((TILE_R, TILE_C), jnp.float32)

    dst = out_ref.at[pl.ds(i * TILE_R, TILE_R), pl.ds(j * TILE_C, TILE_C)]
    overlap = _overlap(sr_ref, sc_ref, step)

    def _compute_path(acc_ref):
        @pl.when((step >= 2) & _overlap(sr_ref, sc_ref, step - 2))
        def _wait_prev():
            pltpu.make_async_copy(acc_ref, dst, sems.at[step - 2]).wait()

        gram = jax.lax.dot_general(
            zi_ref[...], zj_ref[...],
            dimension_numbers=(((1,), (1,)), ((), ())),
            preferred_element_type=jnp.float32,
        )
        mask = kr_ref[...] == kc_ref[...]
        acc_ref[...] = jnp.where(mask, gram, 0.0)
        _fix_diag(acc_ref, i, j)
        pltpu.make_async_copy(acc_ref, dst, sems.at[step]).start()

    @pl.when(overlap & (slot == 0))
    def _s0():
        _compute_path(acc0_ref)

    @pl.when(overlap & (slot == 1))
    def _s1():
        _compute_path(acc1_ref)

    @pl.when(~overlap)
    def _zero():
        pltpu.make_async_copy(zbuf_ref, dst, sems.at[step]).start()

    @pl.when(step == NSTEPS - 1)
    def _drain():
        for k in range(NSTEPS):
            if k + 2 < NSTEPS:
                consumed = (_overlap(sr_ref, sc_ref, k)
                            & _overlap(sr_ref, sc_ref, k + 2))
            else:
                consumed = jnp.bool_(False)

            @pl.when(~consumed)
            def _w():
                pltpu.make_async_copy(zbuf_ref, dst, sems.at[k]).wait()


def kernel(z, cls_label, batch):
    valid = ~((cls_label == 24) | (cls_label == 25) | (cls_label == 26))
    key = batch * 64 + cls_label
    key_row = jnp.where(valid, key, -1).reshape(N, 1)
    key_col = jnp.where(valid, key, -2).reshape(1, N)
    ep_row = jnp.stack([batch[::TILE_R], batch[TILE_R - 1::TILE_R]])
    ep_col = jnp.stack([batch[::TILE_C], batch[TILE_C - 1::TILE_C]])
    grid_spec = pltpu.PrefetchScalarGridSpec(
        num_scalar_prefetch=2,
        grid=(NSTEPS,),
        in_specs=[
            pl.BlockSpec((TILE_R, D), lambda s, sr, sc: (s // GRID_C, 0)),
            pl.BlockSpec((TILE_C, D), lambda s, sr, sc: (s % GRID_C, 0)),
            pl.BlockSpec((TILE_R, 1), lambda s, sr, sc: (s // GRID_C, 0)),
            pl.BlockSpec((1, TILE_C), lambda s, sr, sc: (0, s % GRID_C)),
        ],
        out_specs=pl.BlockSpec(memory_space=pl.ANY),
        scratch_shapes=[
            pltpu.VMEM((TILE_R, TILE_C), jnp.float32),
            pltpu.VMEM((TILE_R, TILE_C), jnp.float32),
            pltpu.VMEM((TILE_R, TILE_C), jnp.float32),
            pltpu.SemaphoreType.DMA((NSTEPS,)),
        ],
    )
    return pl.pallas_call(
        _seg_kernel,
        grid_spec=grid_spec,
        out_shape=jax.ShapeDtypeStruct((N, N), jnp.float32),
    )(ep_row, ep_col, z, z, key_row, key_col)


# speedup vs baseline: 1.0600x; 1.0032x over previous
import jax
import jax.numpy as jnp
from jax.experimental import pallas as pl
from jax.experimental.pallas import tpu as pltpu

N = 4096
D = 128
TILE_R = 2048
TILE_C = 2048
GRID_R = N // TILE_R
GRID_C = N // TILE_C
NSTEPS = GRID_R * GRID_C


def _overlap(sr_ref, sc_ref, step):
    i = step // GRID_C
    j = step % GRID_C
    return (sr_ref[1, i] >= sc_ref[0, j]) & (sc_ref[1, j] >= sr_ref[0, i])


def _fix_diag(acc_ref, i, j):
    off = i * TILE_R - j * TILE_C

    @pl.when((off > -TILE_R) & (off < TILE_C))
    def _():
        eye = (jax.lax.broadcasted_iota(jnp.int32, (128, 128), 0)
               == jax.lax.broadcasted_iota(jnp.int32, (128, 128), 1))
        for k in range(TILE_R // 128):
            c0 = k * 128 + off

            @pl.when((c0 >= 0) & (c0 < TILE_C))
            def _blk():
                rs = pl.ds(k * 128, 128)
                cs = pl.ds(c0, 128)
                acc_ref[rs, cs] = jnp.where(eye, 0.0, acc_ref[rs, cs])


def _seg_kernel(sr_ref, sc_ref, z_ref, kr_ref, kc_ref, out_ref,
                acc0_ref, acc1_ref, zbuf_ref, sems):
    step = pl.program_id(0)
    i = step // GRID_C
    j = step % GRID_C
    slot = jax.lax.rem(step, 2)

    @pl.when(step == 0)
    def _init_zeros():
        zbuf_ref[...] = jnp.zeros((TILE_R, TILE_C), jnp.float32)

    dst = out_ref.at[pl.ds(i * TILE_R, TILE_R), pl.ds(j * TILE_C, TILE_C)]
    overlap = _overlap(sr_ref, sc_ref, step)

    def _compute_path(acc_ref):
        @pl.when((step >= 2) & _overlap(sr_ref, sc_ref, step - 2))
        def _wait_prev():
            pltpu.make_async_copy(acc_ref, dst, sems.at[step - 2]).wait()

        gram = jax.lax.dot_general(
            z_ref[pl.ds(i * TILE_R, TILE_R), :],
            z_ref[pl.ds(j * TILE_C, TILE_C), :],
            dimension_numbers=(((1,), (1,)), ((), ())),
            preferred_element_type=jnp.float32,
        )
        mask = (kr_ref[pl.ds(i * TILE_R, TILE_R), :]
                == kc_ref[:, pl.ds(j * TILE_C, TILE_C)])
        acc_ref[...] = jnp.where(mask, gram, 0.0)
        _fix_diag(acc_ref, i, j)
        pltpu.make_async_copy(acc_ref, dst, sems.at[step]).start()

    @pl.when(overlap & (slot == 0))
    def _s0():
        _compute_path(acc0_ref)

    @pl.when(overlap & (slot == 1))
    def _s1():
        _compute_path(acc1_ref)

    @pl.when(~overlap)
    def _zero():
        pltpu.make_async_copy(zbuf_ref, dst, sems.at[step]).start()

    @pl.when(step == NSTEPS - 1)
    def _drain():
        for k in range(NSTEPS):
            if k + 2 < NSTEPS:
                consumed = (_overlap(sr_ref, sc_ref, k)
                            & _overlap(sr_ref, sc_ref, k + 2))
            else:
                consumed = jnp.bool_(False)

            @pl.when(~consumed)
            def _w():
                pltpu.make_async_copy(zbuf_ref, dst, sems.at[k]).wait()


def kernel(z, cls_label, batch):
    valid = ~((cls_label == 24) | (cls_label == 25) | (cls_label == 26))
    key = batch * 64 + cls_label
    key_row = jnp.where(valid, key, -1).reshape(N, 1)
    key_col = jnp.where(valid, key, -2).reshape(1, N)
    ep_row = jnp.stack([batch[::TILE_R], batch[TILE_R - 1::TILE_R]])
    ep_col = jnp.stack([batch[::TILE_C], batch[TILE_C - 1::TILE_C]])
    grid_spec = pltpu.PrefetchScalarGridSpec(
        num_scalar_prefetch=2,
        grid=(NSTEPS,),
        in_specs=[
            pl.BlockSpec((N, D), lambda s, sr, sc: (0, 0)),
            pl.BlockSpec((N, 1), lambda s, sr, sc: (0, 0)),
            pl.BlockSpec((1, N), lambda s, sr, sc: (0, 0)),
        ],
        out_specs=pl.BlockSpec(memory_space=pl.ANY),
        scratch_shapes=[
            pltpu.VMEM((TILE_R, TILE_C), jnp.float32),
            pltpu.VMEM((TILE_R, TILE_C), jnp.float32),
            pltpu.VMEM((TILE_R, TILE_C), jnp.float32),
            pltpu.SemaphoreType.DMA((NSTEPS,)),
        ],
    )
    return pl.pallas_call(
        _seg_kernel,
        grid_spec=grid_spec,
        out_shape=jax.ShapeDtypeStruct((N, N), jnp.float32),
    )(ep_row, ep_col, z, key_row, key_col)


# speedup vs baseline: 1.2038x; 1.1356x over previous
import jax
import jax.numpy as jnp
from jax.experimental import pallas as pl

N = 4096
D = 128
TILE = 2048


def _seg_kernel(zi_ref, zj_ref, kr_ref, kc_ref, out_ref):
    i = pl.program_id(0)
    j = pl.program_id(1)

    gram = jax.lax.dot_general(
        zi_ref[...], zj_ref[...],
        dimension_numbers=(((1,), (1,)), ((), ())),
        preferred_element_type=jnp.float32,
    )
    mask = kr_ref[...] == kc_ref[...]
    out_ref[...] = jnp.where(mask, gram, 0.0)

    eye = (jax.lax.broadcasted_iota(jnp.int32, (128, 128), 0)
           == jax.lax.broadcasted_iota(jnp.int32, (128, 128), 1))
    diag_tile = i == j
    for k in range(TILE // 128):
        sl = pl.ds(k * 128, 128)
        out_ref[sl, sl] = jnp.where(eye & diag_tile, 0.0, out_ref[sl, sl])


def kernel(z, cls_label, batch):
    valid = ~((cls_label == 24) | (cls_label == 25) | (cls_label == 26))
    key = batch * 64 + cls_label
    key_row = jnp.where(valid, key, -1).reshape(N, 1)
    key_col = jnp.where(valid, key, -2).reshape(1, N)
    grid = (N // TILE, N // TILE)
    return pl.pallas_call(
        _seg_kernel,
        grid=grid,
        in_specs=[
            pl.BlockSpec((TILE, D), lambda i, j: (i, 0)),
            pl.BlockSpec((TILE, D), lambda i, j: (j, 0)),
            pl.BlockSpec((TILE, 1), lambda i, j: (i, 0)),
            pl.BlockSpec((1, TILE), lambda i, j: (0, j)),
        ],
        out_specs=pl.BlockSpec((TILE, TILE), lambda i, j: (i, j)),
        out_shape=jax.ShapeDtypeStruct((N, N), jnp.float32),
    )(z, z, key_row, key_col)
